# baseline (speedup 1.0000x reference)
import jax
import jax.numpy as jnp
from jax.experimental import pallas as pl
from jax.experimental.pallas import tpu as pltpu

D = 768; ED = 256; E = 8; TOPK = 2; DEP = 2; EDEP = 3; NH = 12; DH = 64
V = 20000; P = 16; IMG = 224; FF = 2 * D

BM = 128
N_TOK = 196 + 2048
A = TOPK * N_TOK
L_PAD = ((A + E * (BM - 1) + BM - 1) // BM) * BM
G_TILES = L_PAD // BM


def _ln(x, g, b):
    m = x.mean(-1, keepdims=True)
    v = ((x - m) ** 2).mean(-1, keepdims=True)
    return (x - m) / jnp.sqrt(v + 1e-5) * g + b


def _gn1(x, g, b):
    m = x.mean(axis=(1, 2, 3), keepdims=True)
    v = ((x - m) ** 2).mean(axis=(1, 2, 3), keepdims=True)
    return (x - m) / jnp.sqrt(v + 1e-5) * g[None, :, None, None] + b[None, :, None, None]


def _silu(x):
    return x * jax.nn.sigmoid(x)


def _conv(x, w, b):
    y = jax.lax.conv_general_dilated(x, w, (1, 1), 'SAME',
                                     dimension_numbers=('NCHW', 'OIHW', 'NCHW'))
    return y + b[None, :, None, None]


import functools as _ft


def _moe_tile_kernel(prec, te_ref, xs_ref, w1_ref, b1_ref, w2_ref, b2_ref,
                     wp_ref, out_ref):
    del te_ref
    x = xs_ref[...]
    h = jnp.dot(x, w1_ref[0], preferred_element_type=jnp.float32,
                precision=prec) + b1_ref[0, 0]
    h = h * jax.nn.sigmoid(h)
    y = jnp.dot(h, w2_ref[0], preferred_element_type=jnp.float32,
                precision=prec) + b2_ref[0, 0]
    out_ref[...] = y * wp_ref[...]


def _moe_layer(xt, gate_w, gate_b, w1, b1, w2, b2, prec):
    n = xt.shape[0]
    gl = xt @ gate_w + gate_b
    topv, topi = jax.lax.top_k(gl, TOPK)
    wts = jax.nn.softmax(topv, axis=-1)

    e_flat = topi.reshape(-1).astype(jnp.int32)
    w_flat = wts.reshape(-1)

    aid = jnp.arange(A, dtype=jnp.int32)
    key_sorted = jax.lax.sort(e_flat * 8192 + aid)
    e_sorted = key_sorted >> 13
    a_sorted = key_sorted & 8191
    t_sorted = a_sorted // TOPK
    w_sorted = w_flat[a_sorted]

    counts = jnp.searchsorted(e_sorted, jnp.arange(1, E + 1, dtype=jnp.int32),
                              side='left').astype(jnp.int32)
    grp_off_e = jnp.concatenate([jnp.zeros((1,), jnp.int32), counts[:-1]])
    cnt = counts - grp_off_e
    padded = ((cnt + BM - 1) // BM) * BM
    pad_cum = jnp.cumsum(padded)
    pad_off = pad_cum - padded

    dest = pad_off[e_sorted] + (aid - grp_off_e[e_sorted])
    dpos_sorted = jax.lax.sort(a_sorted * 16384 + dest)
    dpos = (dpos_sorted & 16383).reshape(n, TOPK)

    slots = jnp.arange(L_PAD, dtype=jnp.int32)
    slot_e = jnp.minimum(jnp.searchsorted(pad_cum, slots, side='right'),
                         E - 1).astype(jnp.int32)
    in_grp = slots - pad_off[slot_e]
    valid = in_grp < cnt[slot_e]
    jj = jnp.minimum(grp_off_e[slot_e] + in_grp, A - 1)
    token_pad = jnp.where(valid, t_sorted[jj], 0)
    w_pad = jnp.where(valid, w_sorted[jj], 0.0)

    tile_start = jnp.arange(G_TILES, dtype=jnp.int32) * BM
    tile_e = jnp.searchsorted(pad_cum, tile_start, side='right')
    tile_e = jnp.minimum(tile_e, E - 1).astype(jnp.int32)

    xs_pad = xt[token_pad]

    out_pad = pl.pallas_call(
        _ft.partial(_moe_tile_kernel, prec),
        grid_spec=pltpu.PrefetchScalarGridSpec(
            num_scalar_prefetch=1,
            grid=(G_TILES,),
            in_specs=[
                pl.BlockSpec((BM, D), lambda t, te: (t, 0)),
                pl.BlockSpec((1, D, FF), lambda t, te: (te[t], 0, 0)),
                pl.BlockSpec((1, 1, FF), lambda t, te: (te[t], 0, 0)),
                pl.BlockSpec((1, FF, D), lambda t, te: (te[t], 0, 0)),
                pl.BlockSpec((1, 1, D), lambda t, te: (te[t], 0, 0)),
                pl.BlockSpec((BM, 1), lambda t, te: (t, 0)),
            ],
            out_specs=pl.BlockSpec((BM, D), lambda t, te: (t, 0)),
        ),
        out_shape=jax.ShapeDtypeStruct((L_PAD, D), jnp.float32),
    )(tile_e, xs_pad, w1, b1[:, None, :], w2, b2[:, None, :], w_pad[:, None])

    moe_out = out_pad[dpos[:, 0]] + out_pad[dpos[:, 1]]
    return moe_out


_PREC = [jax.lax.Precision.HIGHEST] * (DEP - 1) + [jax.lax.Precision.DEFAULT]


def kernel(text, img, emb, patch_w, patch_b, enc_ln1_g, enc_ln1_b, enc_wq,
           enc_wk, enc_wv, enc_wo, enc_ln2_g, enc_ln2_b, enc_w1, enc_b1,
           enc_w2, enc_b2, cab_c1_w, cab_c1_b, cab_n1_g, cab_n1_b, cab_c2_w,
           cab_c2_b, cab_n2_g, cab_n2_b, proj_w, proj_b, dec_wq, dec_wk,
           dec_wv, dec_wo, gate_w, gate_b, moe_w1, moe_b1, moe_w2, moe_b2):
    B, S = text.shape
    x_txt = jnp.take(emb, text, axis=0)
    Hs = IMG // P
    n_img = Hs * Hs
    patches = img.reshape(B, 3, Hs, P, Hs, P).transpose(0, 2, 4, 1, 3, 5)
    patches = patches.reshape(B, n_img, 3 * P * P)
    h = patches @ patch_w + patch_b
    eh_heads = 4
    ehd = ED // eh_heads
    for i in range(EDEP):
        hn = _ln(h, enc_ln1_g[i], enc_ln1_b[i])
        q = (hn @ enc_wq[i]).reshape(B, n_img, eh_heads, ehd).transpose(0, 2, 1, 3)
        k = (hn @ enc_wk[i]).reshape(B, n_img, eh_heads, ehd).transpose(0, 2, 1, 3)
        v = (hn @ enc_wv[i]).reshape(B, n_img, eh_heads, ehd).transpose(0, 2, 1, 3)
        a = jax.nn.softmax(q @ k.transpose(0, 1, 3, 2) / jnp.sqrt(float(ehd)), axis=-1)
        o = (a @ v).transpose(0, 2, 1, 3).reshape(B, n_img, ED) @ enc_wo[i]
        h = h + o
        hn2 = _ln(h, enc_ln2_g[i], enc_ln2_b[i])
        h = h + jax.nn.gelu(hn2 @ enc_w1[i] + enc_b1[i]) @ enc_w2[i] + enc_b2[i]
    g = h.transpose(0, 2, 1).reshape(B, ED, Hs, Hs)

    def resblock(z):
        h1 = _silu(_gn1(_conv(z, cab_c1_w, cab_c1_b), cab_n1_g, cab_n1_b))
        h2 = _silu(_gn1(_conv(h1, cab_c2_w, cab_c2_b), cab_n2_g, cab_n2_b))
        return h2 + z

    g = resblock(g)
    g = resblock(g)
    img_tok = g.reshape(B, ED, n_img).transpose(0, 2, 1) @ proj_w + proj_b
    x = jnp.concatenate([img_tok, x_txt], axis=1)
    T = x.shape[1]
    cmask = jnp.tril(jnp.ones((T, T), dtype=bool))
    for i in range(DEP):
        q = (x @ dec_wq[i]).reshape(B, T, NH, DH).transpose(0, 2, 1, 3)
        k = (x @ dec_wk[i]).reshape(B, T, NH, DH).transpose(0, 2, 1, 3)
        v = (x @ dec_wv[i]).reshape(B, T, NH, DH).transpose(0, 2, 1, 3)
        q = q / (jnp.linalg.norm(q, axis=-1, keepdims=True) + 1e-6)
        k = k / (jnp.linalg.norm(k, axis=-1, keepdims=True) + 1e-6)
        logits = (q @ k.transpose(0, 1, 3, 2)) * 10.0
        logits = jnp.where(cmask[None, None, :, :], logits, -1e9)
        a = jax.nn.softmax(logits, axis=-1)
        ao = (a @ v).transpose(0, 2, 1, 3).reshape(B, T, NH * DH) @ dec_wo[i]
        attn = ao + x
        xt = x.reshape(B * T, D)
        moe_out = _moe_layer(xt, gate_w[i], gate_b[i],
                             moe_w1[i], moe_b1[i], moe_w2[i], moe_b2[i],
                             _PREC[i])
        x = attn + moe_out.reshape(B, T, D)
    return x


# speedup vs baseline: 1.2556x; 1.2556x over previous
import jax
import jax.numpy as jnp
from jax.experimental import pallas as pl
from jax.experimental.pallas import tpu as pltpu

D = 768; ED = 256; E = 8; TOPK = 2; DEP = 2; EDEP = 3; NH = 12; DH = 64
V = 20000; P = 16; IMG = 224; FF = 2 * D

BM = 128
N_TOK = 196 + 2048
A = TOPK * N_TOK
L_PAD = ((A + E * (BM - 1) + BM - 1) // BM) * BM
G_TILES = L_PAD // BM


def _ln(x, g, b):
    m = x.mean(-1, keepdims=True)
    v = ((x - m) ** 2).mean(-1, keepdims=True)
    return (x - m) / jnp.sqrt(v + 1e-5) * g + b


def _gn1(x, g, b):
    m = x.mean(axis=(1, 2, 3), keepdims=True)
    v = ((x - m) ** 2).mean(axis=(1, 2, 3), keepdims=True)
    return (x - m) / jnp.sqrt(v + 1e-5) * g[None, :, None, None] + b[None, :, None, None]


def _silu(x):
    return x * jax.nn.sigmoid(x)


def _conv(x, w, b):
    y = jax.lax.conv_general_dilated(x, w, (1, 1), 'SAME',
                                     dimension_numbers=('NCHW', 'OIHW', 'NCHW'))
    return y + b[None, :, None, None]


def _moe_tile_kernel(te_ref, xs_ref, w1_ref, b1_ref, w2_ref, b2_ref, wp_ref,
                     out_ref):
    del te_ref
    x = xs_ref[...]
    h = jnp.dot(x, w1_ref[0], preferred_element_type=jnp.float32) + b1_ref[0, 0]
    h = h * jax.nn.sigmoid(h)
    y = jnp.dot(h, w2_ref[0], preferred_element_type=jnp.float32) + b2_ref[0, 0]
    out_ref[...] = y * wp_ref[...]


def _moe_layer(xt, gate_w, gate_b, w1, b1, w2, b2):
    n = xt.shape[0]
    gl = xt @ gate_w + gate_b
    topv, topi = jax.lax.top_k(gl, TOPK)
    wts = jax.nn.softmax(topv, axis=-1)

    e_flat = topi.reshape(-1).astype(jnp.int32)
    w_flat = wts.reshape(-1)

    key = e_flat * 8192 + jnp.arange(A, dtype=jnp.int32)
    key_sorted = jax.lax.sort(key)
    a_sorted = key_sorted & 8191
    t_sorted = a_sorted // TOPK
    w_sorted = w_flat[a_sorted]

    onehot = (e_flat[:, None] == jnp.arange(E, dtype=jnp.int32)[None, :])
    onehot = onehot.astype(jnp.int32)
    counts = onehot.sum(0)
    rank = ((jnp.cumsum(onehot, axis=0) - onehot) * onehot).sum(1)

    padded = ((counts + BM - 1) // BM) * BM
    pad_cum = jnp.cumsum(padded)
    pad_off = pad_cum - padded
    grp_off = jnp.cumsum(counts) - counts

    slots = jnp.arange(L_PAD, dtype=jnp.int32)
    slot_e = jnp.minimum(jnp.searchsorted(pad_cum, slots, side='right'),
                         E - 1).astype(jnp.int32)
    in_grp = slots - pad_off[slot_e]
    valid = in_grp < counts[slot_e]
    jj = jnp.minimum(grp_off[slot_e] + in_grp, A - 1)
    token_pad = jnp.where(valid, t_sorted[jj], 0)
    w_pad = jnp.where(valid, w_sorted[jj], 0.0)

    dpos = (pad_off[e_flat] + rank).reshape(n, TOPK)

    tile_start = jnp.arange(G_TILES, dtype=jnp.int32) * BM
    tile_e = jnp.searchsorted(pad_cum, tile_start, side='right')
    tile_e = jnp.minimum(tile_e, E - 1).astype(jnp.int32)

    xs_pad = xt[token_pad]

    out_pad = pl.pallas_call(
        _moe_tile_kernel,
        grid_spec=pltpu.PrefetchScalarGridSpec(
            num_scalar_prefetch=1,
            grid=(G_TILES,),
            in_specs=[
                pl.BlockSpec((BM, D), lambda t, te: (t, 0)),
                pl.BlockSpec((1, D, FF), lambda t, te: (te[t], 0, 0)),
                pl.BlockSpec((1, 1, FF), lambda t, te: (te[t], 0, 0)),
                pl.BlockSpec((1, FF, D), lambda t, te: (te[t], 0, 0)),
                pl.BlockSpec((1, 1, D), lambda t, te: (te[t], 0, 0)),
                pl.BlockSpec((BM, 1), lambda t, te: (t, 0)),
            ],
            out_specs=pl.BlockSpec((BM, D), lambda t, te: (t, 0)),
        ),
        out_shape=jax.ShapeDtypeStruct((L_PAD, D), jnp.float32),
    )(tile_e, xs_pad, w1, b1[:, None, :], w2, b2[:, None, :], w_pad[:, None])

    moe_out = out_pad[dpos[:, 0]] + out_pad[dpos[:, 1]]
    return moe_out


def kernel(text, img, emb, patch_w, patch_b, enc_ln1_g, enc_ln1_b, enc_wq,
           enc_wk, enc_wv, enc_wo, enc_ln2_g, enc_ln2_b, enc_w1, enc_b1,
           enc_w2, enc_b2, cab_c1_w, cab_c1_b, cab_n1_g, cab_n1_b, cab_c2_w,
           cab_c2_b, cab_n2_g, cab_n2_b, proj_w, proj_b, dec_wq, dec_wk,
           dec_wv, dec_wo, gate_w, gate_b, moe_w1, moe_b1, moe_w2, moe_b2):
    B, S = text.shape
    x_txt = jnp.take(emb, text, axis=0)
    Hs = IMG // P
    n_img = Hs * Hs
    patches = img.reshape(B, 3, Hs, P, Hs, P).transpose(0, 2, 4, 1, 3, 5)
    patches = patches.reshape(B, n_img, 3 * P * P)
    h = patches @ patch_w + patch_b
    eh_heads = 4
    ehd = ED // eh_heads
    for i in range(EDEP):
        hn = _ln(h, enc_ln1_g[i], enc_ln1_b[i])
        q = (hn @ enc_wq[i]).reshape(B, n_img, eh_heads, ehd).transpose(0, 2, 1, 3)
        k = (hn @ enc_wk[i]).reshape(B, n_img, eh_heads, ehd).transpose(0, 2, 1, 3)
        v = (hn @ enc_wv[i]).reshape(B, n_img, eh_heads, ehd).transpose(0, 2, 1, 3)
        a = jax.nn.softmax(q @ k.transpose(0, 1, 3, 2) / jnp.sqrt(float(ehd)), axis=-1)
        o = (a @ v).transpose(0, 2, 1, 3).reshape(B, n_img, ED) @ enc_wo[i]
        h = h + o
        hn2 = _ln(h, enc_ln2_g[i], enc_ln2_b[i])
        h = h + jax.nn.gelu(hn2 @ enc_w1[i] + enc_b1[i]) @ enc_w2[i] + enc_b2[i]
    g = h.transpose(0, 2, 1).reshape(B, ED, Hs, Hs)

    def resblock(z):
        h1 = _silu(_gn1(_conv(z, cab_c1_w, cab_c1_b), cab_n1_g, cab_n1_b))
        h2 = _silu(_gn1(_conv(h1, cab_c2_w, cab_c2_b), cab_n2_g, cab_n2_b))
        return h2 + z

    g = resblock(g)
    g = resblock(g)
    img_tok = g.reshape(B, ED, n_img).transpose(0, 2, 1) @ proj_w + proj_b
    x = jnp.concatenate([img_tok, x_txt], axis=1)
    T = x.shape[1]
    cmask = jnp.tril(jnp.ones((T, T), dtype=bool))
    for i in range(DEP):
        q = (x @ dec_wq[i]).reshape(B, T, NH, DH).transpose(0, 2, 1, 3)
        k = (x @ dec_wk[i]).reshape(B, T, NH, DH).transpose(0, 2, 1, 3)
        v = (x @ dec_wv[i]).reshape(B, T, NH, DH).transpose(0, 2, 1, 3)
        q = q / (jnp.linalg.norm(q, axis=-1, keepdims=True) + 1e-6)
        k = k / (jnp.linalg.norm(k, axis=-1, keepdims=True) + 1e-6)
        logits = (q @ k.transpose(0, 1, 3, 2)) * 10.0
        logits = jnp.where(cmask[None, None, :, :], logits, -1e9)
        a = jax.nn.softmax(logits, axis=-1)
        ao = (a @ v).transpose(0, 2, 1, 3).reshape(B, T, NH * DH) @ dec_wo[i]
        attn = ao + x
        xt = x.reshape(B * T, D)
        if i < DEP - 1:
            gl = xt @ gate_w[i] + gate_b[i]
            topv, topi = jax.lax.top_k(gl, TOPK)
            w_full = jnp.zeros(gl.shape, gl.dtype).at[
                jnp.arange(xt.shape[0])[:, None], topi].set(
                    jax.nn.softmax(topv, axis=-1))
            ehid = _silu(jnp.einsum('nd,edh->enh', xt, moe_w1[i])
                         + moe_b1[i][:, None, :])
            eout = jnp.einsum('enh,ehd->end', ehid, moe_w2[i]) + moe_b2[i][:, None, :]
            moe_out = jnp.einsum('ne,end->nd', w_full, eout)
        else:
            moe_out = _moe_layer(xt, gate_w[i], gate_b[i],
                                 moe_w1[i], moe_b1[i], moe_w2[i], moe_b2[i])
        x = attn + moe_out.reshape(B, T, D)
    return x
